# Initial kernel scaffold; baseline (speedup 1.0000x reference)
import jax
import jax.numpy as jnp
from jax.experimental import pallas as pl


def kernel(c1, c2, edge_index):
    raise NotImplementedError("write your pallas kernel here")


# speedup vs baseline: 5.8186x; 5.8186x over previous
import functools

import jax
import jax.numpy as jnp
from jax import lax
from jax.experimental import pallas as pl
from jax.experimental.pallas import tpu as pltpu
from jax.experimental.pallas import tpu_sc as plsc

B = 4
N = 50000
D = 128
NC, NS, L = 2, 16, 16
NW = NC * NS
ROWS_PER_W = 1568
NPAD = NW * ROWS_PER_W
TC_BLK = 1568
K = 112
NTILES = ROWS_PER_W // K


def _dense_body(c1_ref, c2_ref, sq_ref, s_ref, d0_ref):
    i = pl.program_id(0)
    d = c2_ref[...] - c1_ref[...]
    row = lax.broadcasted_iota(jnp.int32, (1, TC_BLK, 1), 1) + i * TC_BLK
    d = jnp.where(row < N, d, 0.0)
    s_ref[...] = jnp.sum(d, axis=0)
    d0_ref[...] = d[0]

    @pl.when(i == 0)
    def _():
        sq_ref[...] = jnp.zeros_like(sq_ref)

    sq_ref[...] += jnp.sum(d * d)


_dense_call = pl.pallas_call(
    _dense_body,
    grid=(NPAD // TC_BLK,),
    in_specs=[
        pl.BlockSpec((B, TC_BLK, D), lambda i: (0, i, 0)),
        pl.BlockSpec((B, TC_BLK, D), lambda i: (0, i, 0)),
    ],
    out_specs=[
        pl.BlockSpec((1, 1), lambda i: (0, 0)),
        pl.BlockSpec((TC_BLK, D), lambda i: (i, 0)),
        pl.BlockSpec((TC_BLK, D), lambda i: (i, 0)),
    ],
    out_shape=[
        jax.ShapeDtypeStruct((1, 1), jnp.float32),
        jax.ShapeDtypeStruct((NPAD, D), jnp.float32),
        jax.ShapeDtypeStruct((NPAD, D), jnp.float32),
    ],
)


def _sc_gather_body(d0_hbm, s_hbm, a_hbm, b_hbm, out_hbm,
                    idx_a, idx_b, ra, rb, rs, outv, sem_a, sem_b):
    wid = lax.axis_index("s") * NC + lax.axis_index("c")
    base = wid * ROWS_PER_W

    def tile_body(t, accs):
        a1, a2 = accs
        jb = base + t * K
        pltpu.sync_copy(a_hbm.at[pl.ds(jb, K)], idx_a)
        pltpu.sync_copy(b_hbm.at[pl.ds(jb, K)], idx_b)
        cpa = pltpu.async_copy(d0_hbm.at[idx_a], ra, sem_a)
        cpb = pltpu.async_copy(d0_hbm.at[idx_b], rb, sem_b)
        pltpu.sync_copy(s_hbm.at[pl.ds(jb, K)], rs)
        cpa.wait()
        cpb.wait()

        def row_body(r, racc):
            r1, r2 = racc
            for c in range(D // L):
                va = ra[r, pl.ds(c * L, L)]
                vb = rb[r, pl.ds(c * L, L)]
                vs = rs[r, pl.ds(c * L, L)]
                u = va + vb
                r1 = r1 + u * vs
                r2 = r2 + u * u
            return (r1, r2)

        return lax.fori_loop(0, K, row_body, (a1, a2))

    zero = jnp.zeros((L,), jnp.float32)
    acc1, acc2 = lax.fori_loop(0, NTILES, tile_body, (zero, zero))
    outv[0, :] = acc1
    outv[1, :] = acc2
    pltpu.sync_copy(outv, out_hbm.at[wid])


@functools.cache
def _sc_gather_call():
    mesh = plsc.VectorSubcoreMesh(core_axis_name="c", subcore_axis_name="s")
    return pl.kernel(
        _sc_gather_body,
        out_type=jax.ShapeDtypeStruct((NW, 2, L), jnp.float32),
        mesh=mesh,
        scratch_types=[
            pltpu.VMEM((K,), jnp.int32),
            pltpu.VMEM((K,), jnp.int32),
            pltpu.VMEM((K, D), jnp.float32),
            pltpu.VMEM((K, D), jnp.float32),
            pltpu.VMEM((K, D), jnp.float32),
            pltpu.VMEM((2, L), jnp.float32),
            pltpu.SemaphoreType.DMA,
            pltpu.SemaphoreType.DMA,
        ],
    )


def kernel(c1, c2, edge_index):
    sq, s, d0 = _dense_call(c1, c2)
    dst = edge_index[1].astype(jnp.int32)
    fill = jnp.full((NPAD - N,), N, jnp.int32)
    a_idx = jnp.concatenate([dst[0::2], fill])
    b_idx = jnp.concatenate([dst[1::2], fill])
    partials = _sc_gather_call()(d0, s, a_idx, b_idx)
    acc1 = jnp.sum(partials[:, 0, :])
    acc2 = jnp.sum(partials[:, 1, :])
    return (sq[0, 0] - acc1 + acc2) / (B * N * D)


# speedup vs baseline: 6.8660x; 1.1800x over previous
import functools

import jax
import jax.numpy as jnp
from jax import lax
from jax.experimental import pallas as pl
from jax.experimental.pallas import tpu as pltpu
from jax.experimental.pallas import tpu_sc as plsc

B = 4
N = 50000
D = 128
NC, NS, L = 2, 16, 16
NW = NC * NS
ROWS_PER_W = 1568
NPAD = NW * ROWS_PER_W
TC_BLK = 1568
K = 112
NTILES = ROWS_PER_W // K


def _dense_body(c1_ref, c2_ref, sq_ref, s_ref, d0_ref):
    i = pl.program_id(0)
    d = c2_ref[...] - c1_ref[...]
    row = lax.broadcasted_iota(jnp.int32, (1, TC_BLK, 1), 1) + i * TC_BLK
    d = jnp.where(row < N, d, 0.0)
    s_ref[...] = jnp.sum(d, axis=0)
    d0_ref[...] = d[0]

    @pl.when(i == 0)
    def _():
        sq_ref[...] = jnp.zeros_like(sq_ref)

    sq_ref[...] += jnp.sum(d * d)


_dense_call = pl.pallas_call(
    _dense_body,
    grid=(NPAD // TC_BLK,),
    in_specs=[
        pl.BlockSpec((B, TC_BLK, D), lambda i: (0, i, 0)),
        pl.BlockSpec((B, TC_BLK, D), lambda i: (0, i, 0)),
    ],
    out_specs=[
        pl.BlockSpec((1, 1), lambda i: (0, 0)),
        pl.BlockSpec((TC_BLK, D), lambda i: (i, 0)),
        pl.BlockSpec((TC_BLK, D), lambda i: (i, 0)),
    ],
    out_shape=[
        jax.ShapeDtypeStruct((1, 1), jnp.float32),
        jax.ShapeDtypeStruct((NPAD, D), jnp.float32),
        jax.ShapeDtypeStruct((NPAD, D), jnp.float32),
    ],
)


def _sc_gather_body(d0_hbm, s_hbm, a_hbm, b_hbm, out_hbm,
                    idx_a, idx_b, ra, rb, rs, outv, sems):
    wid = lax.axis_index("s") * NC + lax.axis_index("c")
    base = wid * ROWS_PER_W

    pltpu.sync_copy(a_hbm.at[pl.ds(base, ROWS_PER_W)], idx_a)
    pltpu.sync_copy(b_hbm.at[pl.ds(base, ROWS_PER_W)], idx_b)

    def fire(t):
        buf = t % 2
        jb = base + t * K
        return (
            pltpu.async_copy(d0_hbm.at[idx_a.at[pl.ds(t * K, K)]],
                             ra.at[buf], sems.at[buf, 0]),
            pltpu.async_copy(d0_hbm.at[idx_b.at[pl.ds(t * K, K)]],
                             rb.at[buf], sems.at[buf, 1]),
            pltpu.async_copy(s_hbm.at[pl.ds(jb, K)],
                             rs.at[buf], sems.at[buf, 2]),
        )

    acc1 = jnp.zeros((L,), jnp.float32)
    acc2 = jnp.zeros((L,), jnp.float32)
    handles = {0: fire(0)}
    for t in range(NTILES):
        if t + 1 < NTILES:
            handles[t + 1] = fire(t + 1)
        for h in handles.pop(t):
            h.wait()
        buf = t % 2

        def row_body(r, racc, buf=buf):
            r1, r2 = racc
            for c in range(D // L):
                va = ra[buf, r, pl.ds(c * L, L)]
                vb = rb[buf, r, pl.ds(c * L, L)]
                vs = rs[buf, r, pl.ds(c * L, L)]
                u = va + vb
                r1 = r1 + u * vs
                r2 = r2 + u * u
            return (r1, r2)

        acc1, acc2 = lax.fori_loop(0, K, row_body, (acc1, acc2))

    outv[0, :] = acc1
    outv[1, :] = acc2
    pltpu.sync_copy(outv, out_hbm.at[wid])


@functools.cache
def _sc_gather_call():
    mesh = plsc.VectorSubcoreMesh(core_axis_name="c", subcore_axis_name="s")
    return pl.kernel(
        _sc_gather_body,
        out_type=jax.ShapeDtypeStruct((NW, 2, L), jnp.float32),
        mesh=mesh,
        scratch_types=[
            pltpu.VMEM((ROWS_PER_W,), jnp.int32),
            pltpu.VMEM((ROWS_PER_W,), jnp.int32),
            pltpu.VMEM((2, K, D), jnp.float32),
            pltpu.VMEM((2, K, D), jnp.float32),
            pltpu.VMEM((2, K, D), jnp.float32),
            pltpu.VMEM((2, L), jnp.float32),
            pltpu.SemaphoreType.DMA((2, 3)),
        ],
    )


def kernel(c1, c2, edge_index):
    sq, s, d0 = _dense_call(c1, c2)
    dst = edge_index[1].astype(jnp.int32)
    fill = jnp.full((NPAD - N,), N, jnp.int32)
    a_idx = jnp.concatenate([dst[0::2], fill])
    b_idx = jnp.concatenate([dst[1::2], fill])
    partials = _sc_gather_call()(d0, s, a_idx, b_idx)
    acc1 = jnp.sum(partials[:, 0, :])
    acc2 = jnp.sum(partials[:, 1, :])
    return (sq[0, 0] - acc1 + acc2) / (B * N * D)
